# baseline (speedup 1.0000x reference)
import functools

import jax
import jax.numpy as jnp
from jax import lax
from jax.experimental import pallas as pl
from jax.experimental.pallas import tpu as pltpu
from jax.experimental.pallas import tpu_sc as plsc

N = 10000
E = 320000
D = 128
C = 64

NC, NS = 2, 16
NW = NC * NS
NPAD = 10240
CH = 80
ECH = E // CH
WCH = ECH // NW
RPT = NPAD // NS

_mesh = plsc.VectorSubcoreMesh(core_axis_name="c", subcore_axis_name="s",
                               num_cores=NC, num_subcores=NS)


WCHP = 128
NG = WCHP // 4
NBUF = 4
PREF = 2


def _make_sc_spmm(nhalf):
    qn = C // 16
    NR = N // NS

    @functools.partial(
        pl.kernel,
        out_type=jax.ShapeDtypeStruct((nhalf, NC, NPAD, C), jnp.float32),
        mesh=_mesh,
        scratch_types=[
            pltpu.VMEM((2, NBUF, 2, CH), jnp.int32),
            pltpu.VMEM((NBUF, CH), jnp.int32),
            pltpu.VMEM((NBUF, CH), jnp.int32),
            pltpu.VMEM((NBUF, CH, C), jnp.float32),
            pltpu.VMEM_SHARED((N, C), jnp.float32),
            pltpu.VMEM_SHARED((NPAD, C), jnp.float32),
        ] + [pltpu.SemaphoreType.DMA] * (2 * NBUF + 2),
        compiler_params=pltpu.CompilerParams(needs_layout_passes=False,
                                             use_tc_tiling_on_sc=False),
    )
    def _sc_spmm(*refs):
        ht_hbms = refs[:nhalf]
        (ed_hbm, out_hbm, eslab, sidx, didx, msg, ht_sh, s_sh) = \
            refs[nhalf:nhalf + 8]
        sems = refs[nhalf + 8:]
        gsem = sems[:NBUF]
        ssem = sems[NBUF:2 * NBUF]
        esem = sems[2 * NBUF:]
        cid = lax.axis_index("c")
        sid = lax.axis_index("s")
        wid = sid * NC + cid

        def start_edat(sl, g):
            pltpu.async_copy(ed_hbm.at[wid, g], eslab.at[sl], esem[sl])

        def wait_edat(sl, g):
            pltpu.make_async_copy(ed_hbm.at[wid, g], eslab.at[sl],
                                  esem[sl]).wait()

        def unpack(sl, bb, b):
            for t in range(CH // 16):
                p = eslab[sl, bb, 0, pl.ds(t * 16, 16)]
                sidx[b, pl.ds(t * 16, 16)] = p >> 14
                didx[b, pl.ds(t * 16, 16)] = p & 16383

        def start_gather(b):
            pltpu.async_copy(ht_sh.at[sidx.at[b]], msg.at[b], gsem[b])

        def wait_gather(b):
            pltpu.make_async_copy(ht_sh.at[sidx.at[b]], msg.at[b],
                                  gsem[b]).wait()

        def start_scatter(b):
            pltpu.async_copy(msg.at[b], s_sh.at[didx.at[b]], ssem[b],
                             add=True)

        def wait_scatter(b):
            pltpu.make_async_copy(msg.at[b], s_sh.at[didx.at[b]],
                                  ssem[b]).wait()

        def scale(sl, bb, b):
            def edge4(eo, carry):
                for u in range(4):
                    e = eo * 4 + u
                    wi = plsc.load_gather(
                        eslab, [jnp.full((16,), sl, jnp.int32),
                                jnp.full((16,), bb, jnp.int32),
                                jnp.full((16,), 1, jnp.int32),
                                jnp.full((16,), e, jnp.int32)])
                    w = plsc.bitcast(wi, jnp.float32)
                    for q in range(qn):
                        msg[b, e, pl.ds(q * 16, 16)] = (
                            msg[b, e, pl.ds(q * 16, 16)] * w)
                return carry

            lax.fori_loop(0, CH // 4, edge4, 0)

        def zrow(r, carry):
            for q in range(qn):
                msg[0, r, pl.ds(q * 16, 16)] = jnp.zeros((16,), jnp.float32)
            return carry

        def epoch(half, ht_hbm):
            pltpu.sync_copy(ht_hbm.at[pl.ds(sid * NR, NR)],
                            ht_sh.at[pl.ds(sid * NR, NR)])
            lax.fori_loop(0, CH, zrow, 0)
            for kz in range(RPT // CH):
                pltpu.sync_copy(msg.at[0],
                                s_sh.at[pl.ds(sid * RPT + kz * CH, CH)])
            plsc.subcore_barrier()
            start_edat(0, 0)
            wait_edat(0, 0)
            for b in range(PREF):
                unpack(0, b, b)
                start_gather(b)

            def super_body(sg, carry):
                for gg in range(2):
                    g = sg * 2 + gg
                    sl = gg
                    sl2 = 1 - gg
                    for b in range(NBUF):
                        k = g * NBUF + b
                        if b == 0:
                            @pl.when(g < NG - 1)
                            def _():
                                start_edat(sl2, g + 1)
                        if b == PREF:
                            @pl.when(g < NG - 1)
                            def _():
                                wait_edat(sl2, g + 1)
                        b2 = (b + PREF) % NBUF

                        @pl.when(k >= PREF)
                        def _():
                            wait_scatter(b2)

                        @pl.when(k + PREF < WCHP)
                        def _():
                            if b < NBUF - PREF:
                                unpack(sl, b + PREF, b2)
                            else:
                                unpack(sl2, b + PREF - NBUF, b2)
                            start_gather(b2)

                        wait_gather(b)
                        scale(sl, b, b)
                        start_scatter(b)
                return carry

            lax.fori_loop(0, NG // 2, super_body, 0)
            for b in range(PREF, NBUF):
                wait_scatter(b)
            plsc.subcore_barrier()
            pltpu.sync_copy(s_sh.at[pl.ds(sid * RPT, RPT)],
                            out_hbm.at[half, cid, pl.ds(sid * RPT, RPT)])

        for half in range(nhalf):
            epoch(half, ht_hbms[half])

    return _sc_spmm


_sc_spmm2 = _make_sc_spmm(2)
_sc_spmm1 = _make_sc_spmm(1)


BR = 2000


def _dinv_body(dp_ref, o_ref):
    o_ref[...] = lax.rsqrt(dp_ref[0] + dp_ref[1] + 1.0)


_tc_dinv = pl.pallas_call(
    _dinv_body,
    out_shape=jax.ShapeDtypeStruct((NPAD // 128, 128), jnp.float32),
)


def _pre2_body(h_ref, w_ref, dv_ref, oa_ref, ob_ref):
    hw = jnp.dot(h_ref[...], w_ref[...], preferred_element_type=jnp.float32)
    ht = hw * dv_ref[...]
    oa_ref[...] = ht[:, :C]
    ob_ref[...] = ht[:, C:]


_pre128 = pl.pallas_call(
    _pre2_body,
    grid=(N // BR,),
    in_specs=[pl.BlockSpec((BR, D), lambda i: (i, 0)),
              pl.BlockSpec((D, D), lambda i: (0, 0)),
              pl.BlockSpec((BR, 1), lambda i: (i, 0))],
    out_specs=[pl.BlockSpec((BR, C), lambda i: (i, 0)),
               pl.BlockSpec((BR, C), lambda i: (i, 0))],
    out_shape=[jax.ShapeDtypeStruct((N, C), jnp.float32),
               jax.ShapeDtypeStruct((N, C), jnp.float32)],
)


def _pre1_body(h_ref, w_ref, dv_ref, o_ref):
    hw = jnp.dot(h_ref[...], w_ref[...], preferred_element_type=jnp.float32)
    o_ref[...] = hw * dv_ref[...]


_pre64 = pl.pallas_call(
    _pre1_body,
    grid=(N // BR,),
    in_specs=[pl.BlockSpec((BR, D), lambda i: (i, 0)),
              pl.BlockSpec((D, C), lambda i: (0, 0)),
              pl.BlockSpec((BR, 1), lambda i: (i, 0))],
    out_specs=pl.BlockSpec((BR, C), lambda i: (i, 0)),
    out_shape=jax.ShapeDtypeStruct((N, C), jnp.float32),
)


def _t_full(s_ref, hta_ref, htb_ref, dv_ref, b_ref):
    s = jnp.concatenate([s_ref[0, 0] + s_ref[0, 1] + hta_ref[...],
                         s_ref[1, 0] + s_ref[1, 1] + htb_ref[...]], axis=1)
    return dv_ref[...] * s + b_ref[...]


def _post_relu_body(s_ref, hta_ref, htb_ref, dv_ref, b_ref, o_ref):
    t = _t_full(s_ref, hta_ref, htb_ref, dv_ref, b_ref)
    o_ref[...] = jnp.maximum(t, 0.0)


def _post_norm_body(s_ref, hta_ref, htb_ref, dv_ref, b_ref, id_ref, o_ref):
    t = _t_full(s_ref, hta_ref, htb_ref, dv_ref, b_ref)
    t = jnp.maximum(t, 0.0)
    nrm = jnp.sqrt(jnp.sum(t * t, axis=1, keepdims=True))
    t = t / jnp.maximum(nrm, 1e-12)
    o_ref[...] = (t + id_ref[...]) * 0.5


def _post_lsm_body(s_ref, ht_ref, dv_ref, b_ref, o_ref):
    t = dv_ref[...] * (s_ref[0, 0] + s_ref[0, 1] + ht_ref[...]) + b_ref[...]
    m = jnp.max(t, axis=1, keepdims=True)
    ex = jnp.exp(t - m)
    o_ref[...] = t - m - jnp.log(jnp.sum(ex, axis=1, keepdims=True))


def _make_post(body, extra_in=0):
    specs = [pl.BlockSpec((2, NC, BR, C), lambda i: (0, 0, i, 0)),
             pl.BlockSpec((BR, C), lambda i: (i, 0)),
             pl.BlockSpec((BR, C), lambda i: (i, 0)),
             pl.BlockSpec((BR, 1), lambda i: (i, 0)),
             pl.BlockSpec((1, D), lambda i: (0, 0))]
    specs += [pl.BlockSpec((BR, D), lambda i: (i, 0))] * extra_in
    return pl.pallas_call(
        body,
        grid=(N // BR,),
        in_specs=specs,
        out_specs=pl.BlockSpec((BR, D), lambda i: (i, 0)),
        out_shape=jax.ShapeDtypeStruct((N, D), jnp.float32),
    )


_post_relu = _make_post(_post_relu_body)
_post_norm = _make_post(_post_norm_body, extra_in=1)

_post_lsm = pl.pallas_call(
    _post_lsm_body,
    grid=(N // BR,),
    in_specs=[pl.BlockSpec((1, NC, BR, C), lambda i: (0, 0, i, 0)),
              pl.BlockSpec((BR, C), lambda i: (i, 0)),
              pl.BlockSpec((BR, 1), lambda i: (i, 0)),
              pl.BlockSpec((1, C), lambda i: (0, 0))],
    out_specs=pl.BlockSpec((BR, C), lambda i: (i, 0)),
    out_shape=jax.ShapeDtypeStruct((N, C), jnp.float32),
)


def kernel(x, edge_index, edge_weight, Wb, bb, Wf, bf):
    col2 = edge_index[1].reshape(NW, WCH, CH)
    ew2 = edge_weight.reshape(NW, WCH, CH)
    pk2 = (edge_index[0].reshape(NW, WCH, CH) << 14) | col2
    pad = ((0, 0), (0, WCHP - WCH), (0, 0))
    pkp = jnp.pad(pk2, pad)
    ewi = jax.lax.bitcast_convert_type(jnp.pad(ew2, pad), jnp.int32)
    edat = jnp.stack([pkp, ewi], axis=2).reshape(NW, NG, 4, 2, CH)

    sdeg = _sc_spmm1(jnp.ones((N, C), jnp.float32), edat)
    dinv = _tc_dinv(sdeg[0, :, :, 0].reshape(NC, NPAD // 128, 128))
    dinv = dinv.reshape(NPAD, 1)

    h = x
    for i in range(4):
        identity = h
        hta, htb = _pre128(h, Wb[2 * i], dinv)
        s = _sc_spmm2(hta, htb, edat)
        h = _post_relu(s, hta, htb, dinv, bb[2 * i].reshape(1, D))
        hta, htb = _pre128(h, Wb[2 * i + 1], dinv)
        s = _sc_spmm2(hta, htb, edat)
        h = _post_norm(s, hta, htb, dinv, bb[2 * i + 1].reshape(1, D),
                       identity)

    ht = _pre64(h, Wf, dinv)
    s = _sc_spmm1(ht, edat)
    return _post_lsm(s, ht, dinv, bf.reshape(1, C))


# speedup vs baseline: 1.0443x; 1.0443x over previous
import functools

import jax
import jax.numpy as jnp
from jax import lax
from jax.experimental import pallas as pl
from jax.experimental.pallas import tpu as pltpu
from jax.experimental.pallas import tpu_sc as plsc

N = 10000
E = 320000
D = 128
C = 64

NC, NS = 2, 16
NW = NC * NS
NPAD = 10240
CH = 80
ECH = E // CH
WCH = ECH // NW
RPT = NPAD // NS

_mesh = plsc.VectorSubcoreMesh(core_axis_name="c", subcore_axis_name="s",
                               num_cores=NC, num_subcores=NS)


@functools.partial(
    pl.kernel,
    out_type=jax.ShapeDtypeStruct((NC, NPAD, 16), jnp.float32),
    mesh=_mesh,
    scratch_types=[
        pltpu.VMEM((WCH, CH), jnp.int32),
        pltpu.VMEM((WCH, CH), jnp.float32),
        pltpu.VMEM((CH, 16), jnp.float32),
        pltpu.VMEM((16, 16), jnp.float32),
        pltpu.VMEM_SHARED((NPAD, 16), jnp.float32),
    ],
    compiler_params=pltpu.CompilerParams(needs_layout_passes=False,
                                         use_tc_tiling_on_sc=False),
)
def _sc_deg(col_hbm, ew_hbm, out_hbm, colv, ewv, val, zb, deg_sh):
    cid = lax.axis_index("c")
    sid = lax.axis_index("s")
    wid = sid * NC + cid
    zero16 = jnp.zeros((16,), jnp.float32)
    for r in range(16):
        zb[r, :] = zero16
    for r in range(CH):
        val[r, :] = zero16
    for k in range(RPT // 16):
        pltpu.sync_copy(zb, deg_sh.at[pl.ds(sid * RPT + k * 16, 16)])
    pltpu.sync_copy(col_hbm.at[wid], colv)
    pltpu.sync_copy(ew_hbm.at[wid], ewv)
    plsc.subcore_barrier()
    lane = lax.iota(jnp.int32, 16)
    czero = jnp.zeros((16,), jnp.int32)

    def chunk_body(j, carry):
        for t in range(CH // 16):
            w16 = ewv[j, pl.ds(t * 16, 16)]
            plsc.store_scatter(val, [lane + (t * 16), czero], w16)
        pltpu.sync_copy(val, deg_sh.at[colv.at[j]], add=True)
        return carry

    lax.fori_loop(0, WCH, chunk_body, 0)
    plsc.subcore_barrier()
    pltpu.sync_copy(deg_sh.at[pl.ds(sid * RPT, RPT)],
                    out_hbm.at[cid, pl.ds(sid * RPT, RPT)])


WCHP = 128
NG = WCHP // 4
NBUF = 4
PREF = 2


def _make_sc_spmm(nhalf):
    qn = C // 16
    NR = N // NS

    @functools.partial(
        pl.kernel,
        out_type=jax.ShapeDtypeStruct((nhalf, NC, NPAD, C), jnp.float32),
        mesh=_mesh,
        scratch_types=[
            pltpu.VMEM((2, NBUF, 2, CH), jnp.int32),
            pltpu.VMEM((NBUF, CH), jnp.int32),
            pltpu.VMEM((NBUF, CH), jnp.int32),
            pltpu.VMEM((NBUF, CH, C), jnp.float32),
            pltpu.VMEM_SHARED((N, C), jnp.float32),
            pltpu.VMEM_SHARED((NPAD, C), jnp.float32),
        ] + [pltpu.SemaphoreType.DMA] * (2 * NBUF + 2),
        compiler_params=pltpu.CompilerParams(needs_layout_passes=False,
                                             use_tc_tiling_on_sc=False),
    )
    def _sc_spmm(*refs):
        ht_hbms = refs[:nhalf]
        (ed_hbm, out_hbm, eslab, sidx, didx, msg, ht_sh, s_sh) = \
            refs[nhalf:nhalf + 8]
        sems = refs[nhalf + 8:]
        gsem = sems[:NBUF]
        ssem = sems[NBUF:2 * NBUF]
        esem = sems[2 * NBUF:]
        cid = lax.axis_index("c")
        sid = lax.axis_index("s")
        wid = sid * NC + cid

        def start_edat(sl, g):
            pltpu.async_copy(ed_hbm.at[wid, g], eslab.at[sl], esem[sl])

        def wait_edat(sl, g):
            pltpu.make_async_copy(ed_hbm.at[wid, g], eslab.at[sl],
                                  esem[sl]).wait()

        def unpack(sl, bb, b):
            for t in range(CH // 16):
                p = eslab[sl, bb, 0, pl.ds(t * 16, 16)]
                sidx[b, pl.ds(t * 16, 16)] = p >> 14
                didx[b, pl.ds(t * 16, 16)] = p & 16383

        def start_gather(b):
            pltpu.async_copy(ht_sh.at[sidx.at[b]], msg.at[b], gsem[b])

        def wait_gather(b):
            pltpu.make_async_copy(ht_sh.at[sidx.at[b]], msg.at[b],
                                  gsem[b]).wait()

        def start_scatter(b):
            pltpu.async_copy(msg.at[b], s_sh.at[didx.at[b]], ssem[b],
                             add=True)

        def wait_scatter(b):
            pltpu.make_async_copy(msg.at[b], s_sh.at[didx.at[b]],
                                  ssem[b]).wait()

        def scale(sl, bb, b):
            def edge4(eo, carry):
                for u in range(4):
                    e = eo * 4 + u
                    wi = plsc.load_gather(
                        eslab, [jnp.full((16,), sl, jnp.int32),
                                jnp.full((16,), bb, jnp.int32),
                                jnp.full((16,), 1, jnp.int32),
                                jnp.full((16,), e, jnp.int32)])
                    w = plsc.bitcast(wi, jnp.float32)
                    for q in range(qn):
                        msg[b, e, pl.ds(q * 16, 16)] = (
                            msg[b, e, pl.ds(q * 16, 16)] * w)
                return carry

            lax.fori_loop(0, CH // 4, edge4, 0)

        def zrow(r, carry):
            for q in range(qn):
                msg[0, r, pl.ds(q * 16, 16)] = jnp.zeros((16,), jnp.float32)
            return carry

        def epoch(half, ht_hbm):
            pltpu.sync_copy(ht_hbm.at[pl.ds(sid * NR, NR)],
                            ht_sh.at[pl.ds(sid * NR, NR)])
            lax.fori_loop(0, CH, zrow, 0)
            for kz in range(RPT // CH):
                pltpu.sync_copy(msg.at[0],
                                s_sh.at[pl.ds(sid * RPT + kz * CH, CH)])
            plsc.subcore_barrier()
            start_edat(0, 0)
            wait_edat(0, 0)
            for b in range(PREF):
                unpack(0, b, b)
                start_gather(b)

            def super_body(sg, carry):
                for gg in range(2):
                    g = sg * 2 + gg
                    sl = gg
                    sl2 = 1 - gg
                    for b in range(NBUF):
                        k = g * NBUF + b
                        if b == 0:
                            @pl.when(g < NG - 1)
                            def _():
                                start_edat(sl2, g + 1)
                        if b == PREF:
                            @pl.when(g < NG - 1)
                            def _():
                                wait_edat(sl2, g + 1)
                        b2 = (b + PREF) % NBUF

                        @pl.when(k >= PREF)
                        def _():
                            wait_scatter(b2)

                        @pl.when(k + PREF < WCHP)
                        def _():
                            if b < NBUF - PREF:
                                unpack(sl, b + PREF, b2)
                            else:
                                unpack(sl2, b + PREF - NBUF, b2)
                            start_gather(b2)

                        wait_gather(b)
                        scale(sl, b, b)
                        start_scatter(b)
                return carry

            lax.fori_loop(0, NG // 2, super_body, 0)
            for b in range(PREF, NBUF):
                wait_scatter(b)
            plsc.subcore_barrier()
            pltpu.sync_copy(s_sh.at[pl.ds(sid * RPT, RPT)],
                            out_hbm.at[half, cid, pl.ds(sid * RPT, RPT)])

        for half in range(nhalf):
            epoch(half, ht_hbms[half])

    return _sc_spmm


_sc_spmm2 = _make_sc_spmm(2)
_sc_spmm1 = _make_sc_spmm(1)


BR = 2000


def _dinv_body(dp_ref, o_ref):
    o_ref[...] = lax.rsqrt(dp_ref[0] + dp_ref[1] + 1.0)


_tc_dinv = pl.pallas_call(
    _dinv_body,
    out_shape=jax.ShapeDtypeStruct((NPAD // 128, 128), jnp.float32),
)


def _pre2_body(h_ref, w_ref, dv_ref, oa_ref, ob_ref):
    hw = jnp.dot(h_ref[...], w_ref[...], preferred_element_type=jnp.float32)
    ht = hw * dv_ref[...]
    oa_ref[...] = ht[:, :C]
    ob_ref[...] = ht[:, C:]


_pre128 = pl.pallas_call(
    _pre2_body,
    grid=(N // BR,),
    in_specs=[pl.BlockSpec((BR, D), lambda i: (i, 0)),
              pl.BlockSpec((D, D), lambda i: (0, 0)),
              pl.BlockSpec((BR, 1), lambda i: (i, 0))],
    out_specs=[pl.BlockSpec((BR, C), lambda i: (i, 0)),
               pl.BlockSpec((BR, C), lambda i: (i, 0))],
    out_shape=[jax.ShapeDtypeStruct((N, C), jnp.float32),
               jax.ShapeDtypeStruct((N, C), jnp.float32)],
)


def _pre1_body(h_ref, w_ref, dv_ref, o_ref):
    hw = jnp.dot(h_ref[...], w_ref[...], preferred_element_type=jnp.float32)
    o_ref[...] = hw * dv_ref[...]


_pre64 = pl.pallas_call(
    _pre1_body,
    grid=(N // BR,),
    in_specs=[pl.BlockSpec((BR, D), lambda i: (i, 0)),
              pl.BlockSpec((D, C), lambda i: (0, 0)),
              pl.BlockSpec((BR, 1), lambda i: (i, 0))],
    out_specs=pl.BlockSpec((BR, C), lambda i: (i, 0)),
    out_shape=jax.ShapeDtypeStruct((N, C), jnp.float32),
)


def _t_full(s_ref, hta_ref, htb_ref, dv_ref, b_ref):
    s = jnp.concatenate([s_ref[0, 0] + s_ref[0, 1] + hta_ref[...],
                         s_ref[1, 0] + s_ref[1, 1] + htb_ref[...]], axis=1)
    return dv_ref[...] * s + b_ref[...]


def _post_relu_body(s_ref, hta_ref, htb_ref, dv_ref, b_ref, o_ref):
    t = _t_full(s_ref, hta_ref, htb_ref, dv_ref, b_ref)
    o_ref[...] = jnp.maximum(t, 0.0)


def _post_norm_body(s_ref, hta_ref, htb_ref, dv_ref, b_ref, id_ref, o_ref):
    t = _t_full(s_ref, hta_ref, htb_ref, dv_ref, b_ref)
    t = jnp.maximum(t, 0.0)
    nrm = jnp.sqrt(jnp.sum(t * t, axis=1, keepdims=True))
    t = t / jnp.maximum(nrm, 1e-12)
    o_ref[...] = (t + id_ref[...]) * 0.5


def _post_lsm_body(s_ref, ht_ref, dv_ref, b_ref, o_ref):
    t = dv_ref[...] * (s_ref[0, 0] + s_ref[0, 1] + ht_ref[...]) + b_ref[...]
    m = jnp.max(t, axis=1, keepdims=True)
    ex = jnp.exp(t - m)
    o_ref[...] = t - m - jnp.log(jnp.sum(ex, axis=1, keepdims=True))


def _make_post(body, extra_in=0):
    specs = [pl.BlockSpec((2, NC, BR, C), lambda i: (0, 0, i, 0)),
             pl.BlockSpec((BR, C), lambda i: (i, 0)),
             pl.BlockSpec((BR, C), lambda i: (i, 0)),
             pl.BlockSpec((BR, 1), lambda i: (i, 0)),
             pl.BlockSpec((1, D), lambda i: (0, 0))]
    specs += [pl.BlockSpec((BR, D), lambda i: (i, 0))] * extra_in
    return pl.pallas_call(
        body,
        grid=(N // BR,),
        in_specs=specs,
        out_specs=pl.BlockSpec((BR, D), lambda i: (i, 0)),
        out_shape=jax.ShapeDtypeStruct((N, D), jnp.float32),
    )


_post_relu = _make_post(_post_relu_body)
_post_norm = _make_post(_post_norm_body, extra_in=1)

_post_lsm = pl.pallas_call(
    _post_lsm_body,
    grid=(N // BR,),
    in_specs=[pl.BlockSpec((1, NC, BR, C), lambda i: (0, 0, i, 0)),
              pl.BlockSpec((BR, C), lambda i: (i, 0)),
              pl.BlockSpec((BR, 1), lambda i: (i, 0)),
              pl.BlockSpec((1, C), lambda i: (0, 0))],
    out_specs=pl.BlockSpec((BR, C), lambda i: (i, 0)),
    out_shape=jax.ShapeDtypeStruct((N, C), jnp.float32),
)


def kernel(x, edge_index, edge_weight, Wb, bb, Wf, bf):
    col2 = edge_index[1].reshape(NW, WCH, CH)
    ew2 = edge_weight.reshape(NW, WCH, CH)
    pk2 = (edge_index[0].reshape(NW, WCH, CH) << 14) | col2
    pad = ((0, 0), (0, WCHP - WCH), (0, 0))
    pkp = jnp.pad(pk2, pad)
    ewi = jax.lax.bitcast_convert_type(jnp.pad(ew2, pad), jnp.int32)
    edat = jnp.stack([pkp, ewi], axis=2).reshape(NW, NG, 4, 2, CH)

    degp = _sc_deg(col2, ew2)
    dinv = _tc_dinv(degp[:, :, 0].reshape(NC, NPAD // 128, 128))
    dinv = dinv.reshape(NPAD, 1)

    h = x
    for i in range(4):
        identity = h
        hta, htb = _pre128(h, Wb[2 * i], dinv)
        s = _sc_spmm2(hta, htb, edat)
        h = _post_relu(s, hta, htb, dinv, bb[2 * i].reshape(1, D))
        hta, htb = _pre128(h, Wb[2 * i + 1], dinv)
        s = _sc_spmm2(hta, htb, edat)
        h = _post_norm(s, hta, htb, dinv, bb[2 * i + 1].reshape(1, D),
                       identity)

    ht = _pre64(h, Wf, dinv)
    s = _sc_spmm1(ht, edat)
    return _post_lsm(s, ht, dinv, bf.reshape(1, C))
